# baseline (speedup 1.0000x reference)
import functools

import jax
import jax.numpy as jnp
from jax import lax
from jax.experimental import pallas as pl
from jax.experimental.pallas import tpu as pltpu
from jax.experimental.pallas import tpu_sc as plsc


def _build_pos_kernel(b, d, h, w):
  c2 = 2 * d
  mesh = plsc.VectorSubcoreMesh(core_axis_name="c", subcore_axis_name="s")

  @functools.partial(
      pl.kernel,
      mesh=mesh,
      compiler_params=pltpu.CompilerParams(needs_layout_passes=False),
      out_type=jax.ShapeDtypeStruct((b, h, w, c2), jnp.float32),
      scratch_types=[
          pltpu.VMEM((w, c2), jnp.float32),
          pltpu.VMEM((8, d), jnp.float32),
          pltpu.SemaphoreType.DMA,
      ],
  )
  def pos_kernel(row_hbm, col_hbm, out_hbm, plane_v, rowbuf_v, sem):
    cid = lax.axis_index("c")
    sid = lax.axis_index("s")
    hh = cid * 16 + sid

    pltpu.sync_copy(col_hbm.at[pl.ds(0, w), :], plane_v.at[:, pl.ds(0, d)])
    h8 = (hh // 8) * 8
    pltpu.sync_copy(row_hbm.at[pl.ds(h8, 8), :], rowbuf_v)
    hrow = hh - h8
    vecs = [rowbuf_v[hrow, pl.ds(16 * k, 16)] for k in range(d // 16)]

    def fill(w1, carry):
      for k, v in enumerate(vecs):
        plane_v[w1, pl.ds(d + 16 * k, 16)] = v
      return carry

    lax.fori_loop(0, w, fill, 0)

    handles = []
    for bb in range(b):
      handles.append(pltpu.async_copy(plane_v, out_hbm.at[bb, hh], sem))
    for hd in handles:
      hd.wait()

  return pos_kernel


def kernel(x, row_embed, col_embed):
  b = x.shape[0]
  h = x.shape[-2]
  w = x.shape[-1]
  d = col_embed.shape[-1]
  out_phys = _build_pos_kernel(b, d, h, w)(row_embed, col_embed)
  return jnp.transpose(out_phys, (0, 3, 1, 2))


# speedup vs baseline: 1.0418x; 1.0418x over previous
import functools

import jax
import jax.numpy as jnp
from jax import lax
from jax.experimental import pallas as pl
from jax.experimental.pallas import tpu as pltpu
from jax.experimental.pallas import tpu_sc as plsc


def _build_pos_kernel(b, d, h, w):
  c2 = 2 * d
  mesh = plsc.VectorSubcoreMesh(core_axis_name="c", subcore_axis_name="s")

  @functools.partial(
      pl.kernel,
      mesh=mesh,
      compiler_params=pltpu.CompilerParams(needs_layout_passes=False),
      out_type=jax.ShapeDtypeStruct((b, h, w, c2), jnp.float32),
      scratch_types=[
          pltpu.VMEM((w, c2), jnp.float32),
          pltpu.VMEM((8, d), jnp.float32),
          pltpu.SemaphoreType.DMA,
      ],
  )
  def pos_kernel(row_hbm, col_hbm, out_hbm, plane_v, rowbuf_v, sem):
    cid = lax.axis_index("c")
    sid = lax.axis_index("s")
    hh = sid * 2 + cid

    pltpu.sync_copy(col_hbm.at[pl.ds(0, w), :], plane_v.at[:, pl.ds(0, d)])
    h8 = (hh // 8) * 8
    pltpu.sync_copy(row_hbm.at[pl.ds(h8, 8), :], rowbuf_v)
    hrow = hh - h8
    vecs = [rowbuf_v[hrow, pl.ds(16 * k, 16)] for k in range(d // 16)]

    def fill(w1, carry):
      for k, v in enumerate(vecs):
        plane_v[w1, pl.ds(d + 16 * k, 16)] = v
      return carry

    lax.fori_loop(0, w, fill, 0)

    handles = []
    for bb in range(b):
      handles.append(pltpu.async_copy(plane_v, out_hbm.at[bb, hh], sem))
    for hd in handles:
      hd.wait()

  return pos_kernel


def kernel(x, row_embed, col_embed):
  b = x.shape[0]
  h = x.shape[-2]
  w = x.shape[-1]
  d = col_embed.shape[-1]
  out_phys = _build_pos_kernel(b, d, h, w)(row_embed, col_embed)
  return jnp.transpose(out_phys, (0, 3, 1, 2))
